# baseline (speedup 1.0000x reference)
import functools

import jax
import jax.numpy as jnp
from jax import lax
from jax.experimental import pallas as pl
from jax.experimental.pallas import tpu as pltpu
from jax.experimental.pallas import tpu_sc as plsc

D = 128
N1, N2, N3, N4 = 100000, 50000, 25000, 12500


def _silu(x):
    return x * jax.nn.sigmoid(x)


def _mlp(x, wa, ba, wb, bb):
    h = _silu(jnp.dot(x, wa, preferred_element_type=jnp.float32) + ba)
    return jnp.dot(h, wb, preferred_element_type=jnp.float32) + bb


def _store_chunks(refs, val, cw):
    for c, r in enumerate(refs):
        r[...] = val[:, c * cw:(c + 1) * cw]


def _cat(refs):
    return jnp.concatenate([r[...] for r in refs], axis=1)


def _recon_from_ls(ls):
    l, p = ls[:, :D], ls[:, D:]
    digs = []
    for _ in range(4):
        d = p - 64.0 * jnp.floor(p * (1.0 / 64.0))
        digs.append(d)
        p = jnp.floor(p * (1.0 / 64.0))
    s = jnp.concatenate(digs, axis=1)
    par = s - 2.0 * jnp.floor(s * 0.5)
    return jnp.exp(l) * (1.0 - 2.0 * par)


def _log_sign(h):
    s = jnp.where(h < 0, 1.0, 0.0).astype(jnp.float32)
    packed = (s[:, 0:32] + 64.0 * s[:, 32:64] + 4096.0 * s[:, 64:96]
              + 262144.0 * s[:, 96:128])
    return jnp.concatenate([jnp.log(jnp.abs(h)), packed], axis=-1)


def _mlp1_body(x_ref, wa_ref, ba_ref, wb_ref, bb_ref, h_ref, *ls_refs):
    h = _mlp(x_ref[...], wa_ref[...], ba_ref[...], wb_ref[...], bb_ref[...])
    h_ref[...] = h
    _store_chunks(ls_refs, _log_sign(h), 16)


def _mlp2_body(*refs):
    lss_refs, (wa, ba, wb, bb), h_ref, ls_refs = (
        refs[:10], refs[10:14], refs[14], refs[15:])
    x = _recon_from_ls(_cat(lss_refs))
    h = _mlp(x, wa[...], ba[...], wb[...], bb[...])
    h_ref[...] = h
    _store_chunks(ls_refs, _log_sign(h), 16)


def _mlp3_body(*refs):
    lss_refs, (wa, ba, wb, bb), ls_refs = refs[:10], refs[10:14], refs[14:]
    x = _recon_from_ls(_cat(lss_refs))
    h = _mlp(x, wa[...], ba[...], wb[...], bb[...])
    _store_chunks(ls_refs, _log_sign(h), 16)


def _recon_body(*refs):
    lss_refs, h_refs = refs[:10], refs[10:]
    _store_chunks(h_refs, _recon_from_ls(_cat(lss_refs)), 16)


def _m2_body(*refs):
    h2_ref, hd_refs, (wb, wc), o_refs = (
        refs[0], refs[1:9], refs[9:11], refs[11:])
    acc = jnp.dot(h2_ref[...], wb[...], preferred_element_type=jnp.float32)
    acc += jnp.dot(_cat(hd_refs), wc[...], preferred_element_type=jnp.float32)
    _store_chunks(o_refs, acc, 16)


def _final_body(*refs):
    h1_ref, q_refs, wa, b_ref, o_ref = (
        refs[0], refs[1:9], refs[9], refs[10], refs[11])
    acc = jnp.dot(h1_ref[...], wa[...], preferred_element_type=jnp.float32)
    o_ref[...] = _silu(acc + _cat(q_refs) + b_ref[...])


def _tc_call(body, n, bn, in_widths, full_shapes, out_widths, args):
    grid = n // bn
    in_specs = ([pl.BlockSpec((bn, w), lambda i: (i, 0)) for w in in_widths]
                + [pl.BlockSpec(s, lambda i: (0, 0)) for s in full_shapes])
    out_specs = [pl.BlockSpec((bn, w), lambda i: (i, 0)) for w in out_widths]
    out_shape = [jax.ShapeDtypeStruct((n, w), jnp.float32) for w in out_widths]
    single = len(out_widths) == 1
    if single:
        out_specs, out_shape = out_specs[0], out_shape[0]
    res = pl.pallas_call(
        body, grid=(grid,), in_specs=in_specs, out_specs=out_specs,
        out_shape=out_shape,
        compiler_params=pltpu.CompilerParams(
            dimension_semantics=("parallel",)),
    )(*args)
    return res


def _ceil_to(x, m):
    return (x + m - 1) // m * m


NBUF = 4
ZB = 256


@functools.lru_cache(maxsize=None)
def _make_segsum(nv, nout, e, nch, cw, halves=1):
    e_pad = _ceil_to(e, 16 * 128 * NBUF)
    np_half = _ceil_to((nout + halves) // halves, 2048)
    np_pad = np_half * halves
    epw = e_pad // 16
    nb = epw // 128
    ng = nb // NBUF
    zrows = np_half // 16
    mesh = plsc.VectorSubcoreMesh(core_axis_name="c", subcore_axis_name="s")

    @functools.partial(
        pl.kernel,
        out_type=[jax.ShapeDtypeStruct((np_pad, cw), jnp.float32)
                  for _ in range(nch)],
        mesh=mesh,
        compiler_params=pltpu.CompilerParams(use_tc_tiling_on_sc=False),
        scratch_types=[
            pltpu.VMEM((epw,), jnp.int32),
            pltpu.VMEM((epw,), jnp.int32),
            pltpu.VMEM((epw,), jnp.int32),
            [pltpu.VMEM((128, cw), jnp.float32) for _ in range(NBUF)],
            pltpu.VMEM((ZB, cw), jnp.float32),
            pltpu.VMEM_SHARED((np_half + 16, cw), jnp.float32),
            pltpu.SemaphoreType.DMA((NBUF,)),
            pltpu.SemaphoreType.DMA((NBUF,)),
        ],
    )
    def seg_kernel(*refs):
        tbls = refs[:nch]
        g_hbm, s_hbm, zeros_hbm = refs[nch:nch + 3]
        outs = refs[nch + 3:2 * nch + 3]
        gb, sb, sb2, rows, zbuf, acc, gsem, ssem = refs[2 * nch + 3:]
        cid = lax.axis_index("c")
        sid = lax.axis_index("s")

        pltpu.sync_copy(zeros_hbm, zbuf)
        pltpu.sync_copy(g_hbm.at[pl.ds(sid * epw, epw)], gb)
        pltpu.sync_copy(s_hbm.at[pl.ds(sid * epw, epw)], sb)

        def one_chunk(tbl, out, hv):
            if halves > 1:
                base = hv * np_half

                def remap(r, _):
                    v = sb[pl.ds(r * 16, 16)] - base
                    inb = (v >= 0) & (v < np_half)
                    sb2[pl.ds(r * 16, 16)] = jnp.where(inb, v, np_half)
                    return _
                lax.fori_loop(0, epw // 16, remap, None)
            sbx = sb2 if halves > 1 else sb

            lo = sid * zrows
            for z0 in range(0, zrows, ZB):
                zn = min(ZB, zrows - z0)
                pltpu.sync_copy(zbuf.at[pl.ds(0, zn)],
                                acc.at[pl.ds(lo + z0, zn)])
            plsc.subcore_barrier()

            def g_start(j, b):
                pltpu.async_copy(tbl.at[gb.at[pl.ds(j * 128, 128)]],
                                 rows[b], gsem.at[b])

            def g_wait(j, b):
                pltpu.make_async_copy(tbl.at[gb.at[pl.ds(j * 128, 128)]],
                                      rows[b], gsem.at[b]).wait()

            def s_start(j, b):
                pltpu.async_copy(rows[b], acc.at[sbx.at[pl.ds(j * 128, 128)]],
                                 ssem.at[b], add=True)

            def s_wait(j, b):
                pltpu.make_async_copy(rows[b],
                                      acc.at[sbx.at[pl.ds(j * 128, 128)]],
                                      ssem.at[b]).wait()

            for b in range(NBUF):
                g_start(b, b)
            for b in range(NBUF):
                g_wait(b, b)
                s_start(b, b)

            def round_body(g, _):
                j0 = g * NBUF
                for b in range(NBUF):
                    s_wait(j0 - NBUF + b, b)
                    g_start(j0 + b, b)
                for b in range(NBUF):
                    g_wait(j0 + b, b)
                    s_start(j0 + b, b)
                return _
            lax.fori_loop(1, ng, round_body, None)
            for b in range(NBUF):
                s_wait((ng - 1) * NBUF + b, b)
            plsc.subcore_barrier()

            pltpu.sync_copy(acc.at[pl.ds(lo, zrows)],
                            out.at[pl.ds(hv * np_half + lo, zrows)])
            plsc.subcore_barrier()

        for p in range(nch // 2):
            for c in range(2):
                k = p * 2 + c

                @pl.when(cid == c)
                def _run(k=k):
                    for hv in range(halves):
                        one_chunk(tbls[k], outs[k], hv)

    return seg_kernel


def _segsum(tbls, g, s, nout, cw, halves=1):
    nv = tbls[0].shape[0]
    nch = len(tbls)
    e = g.shape[0]
    e_pad = _ceil_to(e, 16 * 128 * NBUF)
    g = jnp.concatenate([g, jnp.zeros((e_pad - e,), jnp.int32)])
    s = jnp.concatenate([s, jnp.full((e_pad - e,), nout, jnp.int32)])
    zeros = jnp.zeros((ZB, cw), jnp.float32)
    fn = _make_segsum(nv, nout, e, nch, cw, halves)
    return fn(*tbls, g, s, zeros)


def kernel(feat, src12, dst12, src23, dst23, src34, dst34,
           W1a, b1a, W1b, b1b, W2a, b2a, W2b, b2b, W3a, b3a, W3b, b3b, W, b):
    b1a_, b1b_ = b1a.reshape(1, D), b1b.reshape(1, D)
    b2a_, b2b_ = b2a.reshape(1, D), b2b.reshape(1, D)
    b3a_, b3b_ = b3a.reshape(1, D), b3b.reshape(1, D)
    b_ = b.reshape(1, D)

    mlp1_out = _tc_call(_mlp1_body, N1, 1000, [D],
                        [(D, D), (1, D), (D, D), (1, D)],
                        [D] + [16] * 10,
                        (feat, W1a, b1a_, W1b, b1b_))
    h1, ls1 = mlp1_out[0], mlp1_out[1:]
    lss2 = _segsum(ls1, src12, dst12, N2, 16)
    mlp2_out = _tc_call(_mlp2_body, N2, 1000, [16] * 10,
                        [(D, D), (1, D), (D, D), (1, D)],
                        [D] + [16] * 10,
                        (*lss2, W2a, b2a_, W2b, b2b_))
    h2, ls2 = mlp2_out[0], mlp2_out[1:]
    lss3 = _segsum(ls2, src23, dst23, N3, 16)
    ls3 = _tc_call(_mlp3_body, N3, 1000, [16] * 10,
                   [(D, D), (1, D), (D, D), (1, D)],
                   [16] * 10,
                   (*lss3, W3a, b3a_, W3b, b3b_))
    lss4 = _segsum(ls3, src34, dst34, N4, 16)
    np4 = lss4[0].shape[0]
    h4 = _tc_call(_recon_body, np4, np4 // 8, [16] * 10, [], [16] * 8, lss4)

    h3_d = _segsum(h4, dst34, src34, N3, 16)
    h2_d = _segsum(h3_d, dst23, src23, N2, 16)
    Wa, Wb, Wc = W[:D], W[D:2 * D], W[2 * D:]
    m2 = _tc_call(_m2_body, N2, 1000, [D] + [16] * 8,
                  [(D, D), (D, D)], [16] * 8,
                  (h2, *h2_d, Wb, Wc))
    q = _segsum(m2, dst12, src12, N1, 16, halves=2)

    out = _tc_call(_final_body, N1, 1000, [D] + [16] * 8,
                   [(D, D), (1, D)], [D],
                   (h1, *q, Wa, b_))
    return out


# speedup vs baseline: 1.0915x; 1.0915x over previous
import functools

import jax
import jax.numpy as jnp
from jax import lax
from jax.experimental import pallas as pl
from jax.experimental.pallas import tpu as pltpu
from jax.experimental.pallas import tpu_sc as plsc

D = 128
N1, N2, N3, N4 = 100000, 50000, 25000, 12500


def _silu(x):
    return x * jax.nn.sigmoid(x)


def _mlp(x, wa, ba, wb, bb):
    h = _silu(jnp.dot(x, wa, preferred_element_type=jnp.float32) + ba)
    return jnp.dot(h, wb, preferred_element_type=jnp.float32) + bb


def _store_chunks(refs, val, cw):
    for c, r in enumerate(refs):
        r[...] = val[:, c * cw:(c + 1) * cw]


def _cat(refs):
    return jnp.concatenate([r[...] for r in refs], axis=1)


def _recon_from_ls(ls):
    l, p = ls[:, :D], ls[:, D:]
    digs = []
    for _ in range(4):
        d = p - 64.0 * jnp.floor(p * (1.0 / 64.0))
        digs.append(d)
        p = jnp.floor(p * (1.0 / 64.0))
    s = jnp.concatenate(digs, axis=1)
    par = s - 2.0 * jnp.floor(s * 0.5)
    return jnp.exp(l) * (1.0 - 2.0 * par)


def _log_sign(h):
    s = jnp.where(h < 0, 1.0, 0.0).astype(jnp.float32)
    packed = (s[:, 0:32] + 64.0 * s[:, 32:64] + 4096.0 * s[:, 64:96]
              + 262144.0 * s[:, 96:128])
    return jnp.concatenate([jnp.log(jnp.abs(h)), packed], axis=-1)


def _mlp1_body(x_ref, wa_ref, ba_ref, wb_ref, bb_ref, h_ref, *ls_refs):
    h = _mlp(x_ref[...], wa_ref[...], ba_ref[...], wb_ref[...], bb_ref[...])
    h_ref[...] = h
    _store_chunks(ls_refs, _log_sign(h), 16)


def _mlp2_body(*refs):
    lss_refs, (wa, ba, wb, bb), h_ref, ls_refs = (
        refs[:10], refs[10:14], refs[14], refs[15:])
    x = _recon_from_ls(_cat(lss_refs))
    h = _mlp(x, wa[...], ba[...], wb[...], bb[...])
    h_ref[...] = h
    _store_chunks(ls_refs, _log_sign(h), 16)


def _mlp3_body(*refs):
    lss_refs, (wa, ba, wb, bb), ls_refs = refs[:10], refs[10:14], refs[14:]
    x = _recon_from_ls(_cat(lss_refs))
    h = _mlp(x, wa[...], ba[...], wb[...], bb[...])
    _store_chunks(ls_refs, _log_sign(h), 16)


def _recon_body(*refs):
    lss_refs, h_refs = refs[:10], refs[10:]
    _store_chunks(h_refs, _recon_from_ls(_cat(lss_refs)), 16)


def _m2_body(*refs):
    h2_ref, hd_refs, (wb, wc), o_refs = (
        refs[0], refs[1:9], refs[9:11], refs[11:])
    acc = jnp.dot(h2_ref[...], wb[...], preferred_element_type=jnp.float32)
    acc += jnp.dot(_cat(hd_refs), wc[...], preferred_element_type=jnp.float32)
    _store_chunks(o_refs, acc, 8)


def _final_body(*refs):
    h1_ref, q_refs, wa, b_ref, o_ref = (
        refs[0], refs[1:17], refs[17], refs[18], refs[19])
    acc = jnp.dot(h1_ref[...], wa[...], preferred_element_type=jnp.float32)
    o_ref[...] = _silu(acc + _cat(q_refs) + b_ref[...])


def _tc_call(body, n, bn, in_widths, full_shapes, out_widths, args):
    grid = n // bn
    in_specs = ([pl.BlockSpec((bn, w), lambda i: (i, 0)) for w in in_widths]
                + [pl.BlockSpec(s, lambda i: (0, 0)) for s in full_shapes])
    out_specs = [pl.BlockSpec((bn, w), lambda i: (i, 0)) for w in out_widths]
    out_shape = [jax.ShapeDtypeStruct((n, w), jnp.float32) for w in out_widths]
    single = len(out_widths) == 1
    if single:
        out_specs, out_shape = out_specs[0], out_shape[0]
    res = pl.pallas_call(
        body, grid=(grid,), in_specs=in_specs, out_specs=out_specs,
        out_shape=out_shape,
        compiler_params=pltpu.CompilerParams(
            dimension_semantics=("parallel",)),
    )(*args)
    return res


def _ceil_to(x, m):
    return (x + m - 1) // m * m


NBUF = 4
ZB = 256


@functools.lru_cache(maxsize=None)
def _make_segsum(nv, nout, e, nch, cw):
    e_pad = _ceil_to(e, 16 * 128 * NBUF)
    np_pad = _ceil_to(nout + 1, 2048)
    epw = e_pad // 16
    nb = epw // 128
    ng = nb // NBUF
    zrows = np_pad // 16
    mesh = plsc.VectorSubcoreMesh(core_axis_name="c", subcore_axis_name="s")

    @functools.partial(
        pl.kernel,
        out_type=[jax.ShapeDtypeStruct((np_pad, cw), jnp.float32)
                  for _ in range(nch)],
        mesh=mesh,
        compiler_params=pltpu.CompilerParams(use_tc_tiling_on_sc=False),
        scratch_types=[
            pltpu.VMEM((epw,), jnp.int32),
            pltpu.VMEM((epw,), jnp.int32),
            [pltpu.VMEM((128, cw), jnp.float32) for _ in range(NBUF)],
            pltpu.VMEM((ZB, cw), jnp.float32),
            pltpu.VMEM_SHARED((np_pad, cw), jnp.float32),
            pltpu.SemaphoreType.DMA((NBUF,)),
            pltpu.SemaphoreType.DMA((NBUF,)),
        ],
    )
    def seg_kernel(*refs):
        tbls = refs[:nch]
        g_hbm, s_hbm, zeros_hbm = refs[nch:nch + 3]
        outs = refs[nch + 3:2 * nch + 3]
        gb, sb, rows, zbuf, acc, gsem, ssem = refs[2 * nch + 3:]
        cid = lax.axis_index("c")
        sid = lax.axis_index("s")

        pltpu.sync_copy(zeros_hbm, zbuf)
        pltpu.sync_copy(g_hbm.at[pl.ds(sid * epw, epw)], gb)
        pltpu.sync_copy(s_hbm.at[pl.ds(sid * epw, epw)], sb)

        def one_chunk(tbl, out):
            lo = sid * zrows
            for z0 in range(0, zrows, ZB):
                zn = min(ZB, zrows - z0)
                pltpu.sync_copy(zbuf.at[pl.ds(0, zn)],
                                acc.at[pl.ds(lo + z0, zn)])
            plsc.subcore_barrier()

            def g_start(j, b):
                pltpu.async_copy(tbl.at[gb.at[pl.ds(j * 128, 128)]],
                                 rows[b], gsem.at[b])

            def g_wait(j, b):
                pltpu.make_async_copy(tbl.at[gb.at[pl.ds(j * 128, 128)]],
                                      rows[b], gsem.at[b]).wait()

            def s_start(j, b):
                pltpu.async_copy(rows[b], acc.at[sb.at[pl.ds(j * 128, 128)]],
                                 ssem.at[b], add=True)

            def s_wait(j, b):
                pltpu.make_async_copy(rows[b],
                                      acc.at[sb.at[pl.ds(j * 128, 128)]],
                                      ssem.at[b]).wait()

            for b in range(NBUF):
                g_start(b, b)
            for b in range(NBUF):
                g_wait(b, b)
                s_start(b, b)

            def round_body(g, _):
                j0 = g * NBUF
                for b in range(NBUF):
                    s_wait(j0 - NBUF + b, b)
                    g_start(j0 + b, b)
                for b in range(NBUF):
                    g_wait(j0 + b, b)
                    s_start(j0 + b, b)
                return _
            lax.fori_loop(1, ng, round_body, None)
            for b in range(NBUF):
                s_wait((ng - 1) * NBUF + b, b)
            plsc.subcore_barrier()

            pltpu.sync_copy(acc.at[pl.ds(lo, zrows)],
                            out.at[pl.ds(lo, zrows)])
            plsc.subcore_barrier()

        for p in range(nch // 2):
            for c in range(2):
                k = p * 2 + c

                @pl.when(cid == c)
                def _run(k=k):
                    one_chunk(tbls[k], outs[k])

    return seg_kernel


def _segsum(tbls, g, s, nout, cw):
    nv = tbls[0].shape[0]
    nch = len(tbls)
    e = g.shape[0]
    e_pad = _ceil_to(e, 16 * 128 * NBUF)
    g = jnp.concatenate([g, jnp.zeros((e_pad - e,), jnp.int32)])
    s = jnp.concatenate([s, jnp.full((e_pad - e,), nout, jnp.int32)])
    zeros = jnp.zeros((ZB, cw), jnp.float32)
    fn = _make_segsum(nv, nout, e, nch, cw)
    return fn(*tbls, g, s, zeros)


def kernel(feat, src12, dst12, src23, dst23, src34, dst34,
           W1a, b1a, W1b, b1b, W2a, b2a, W2b, b2b, W3a, b3a, W3b, b3b, W, b):
    b1a_, b1b_ = b1a.reshape(1, D), b1b.reshape(1, D)
    b2a_, b2b_ = b2a.reshape(1, D), b2b.reshape(1, D)
    b3a_, b3b_ = b3a.reshape(1, D), b3b.reshape(1, D)
    b_ = b.reshape(1, D)

    mlp1_out = _tc_call(_mlp1_body, N1, 1000, [D],
                        [(D, D), (1, D), (D, D), (1, D)],
                        [D] + [16] * 10,
                        (feat, W1a, b1a_, W1b, b1b_))
    h1, ls1 = mlp1_out[0], mlp1_out[1:]
    lss2 = _segsum(ls1, src12, dst12, N2, 16)
    mlp2_out = _tc_call(_mlp2_body, N2, 1000, [16] * 10,
                        [(D, D), (1, D), (D, D), (1, D)],
                        [D] + [16] * 10,
                        (*lss2, W2a, b2a_, W2b, b2b_))
    h2, ls2 = mlp2_out[0], mlp2_out[1:]
    lss3 = _segsum(ls2, src23, dst23, N3, 16)
    ls3 = _tc_call(_mlp3_body, N3, 1000, [16] * 10,
                   [(D, D), (1, D), (D, D), (1, D)],
                   [16] * 10,
                   (*lss3, W3a, b3a_, W3b, b3b_))
    lss4 = _segsum(ls3, src34, dst34, N4, 16)
    np4 = lss4[0].shape[0]
    h4 = _tc_call(_recon_body, np4, np4 // 8, [16] * 10, [], [16] * 8, lss4)

    h3_d = _segsum(h4, dst34, src34, N3, 16)
    h2_d = _segsum(h3_d, dst23, src23, N2, 16)
    Wa, Wb, Wc = W[:D], W[D:2 * D], W[2 * D:]
    m2 = _tc_call(_m2_body, N2, 1000, [D] + [16] * 8,
                  [(D, D), (D, D)], [8] * 16,
                  (h2, *h2_d, Wb, Wc))
    q = _segsum(m2, dst12, src12, N1, 8)

    out = _tc_call(_final_body, N1, 1000, [D] + [8] * 16,
                   [(D, D), (1, D)], [D],
                   (h1, *q, Wa, b_))
    return out
